# baseline (speedup 1.0000x reference)
import jax
import jax.numpy as jnp
from jax import lax
from jax.experimental import pallas as pl
from jax.experimental.pallas import tpu as pltpu
from jax.experimental.pallas import tpu_sc as plsc

D = 1024
NC = 2
NS = 16
NW = NC * NS
B = 4 * 8192
BPW = B // NW
C = 48
CHUNKS = [(i * C, C) for i in range(BPW // C)] + [(BPW - BPW % C, BPW % C)]
NCH = len(CHUNKS)
NBUF = 2


def _pe_body(idx_hbm, table_hbm, out_hbm, idx_v, rows_v, gsem, osem):
    wid = lax.axis_index("s") * NC + lax.axis_index("c")
    base = wid * BPW
    pltpu.sync_copy(idx_hbm.at[wid], idx_v)

    def start_gather(j):
        off, sz = CHUNKS[j]
        slot = j % NBUF
        dst = rows_v.at[slot] if sz == C else rows_v.at[slot].at[pl.ds(0, sz)]
        return pltpu.async_copy(
            table_hbm.at[idx_v.at[pl.ds(off, sz)]], dst, gsem)

    def start_out(j):
        off, sz = CHUNKS[j]
        slot = j % NBUF
        src = rows_v.at[slot] if sz == C else rows_v.at[slot].at[pl.ds(0, sz)]
        return pltpu.async_copy(
            src, out_hbm.at[pl.ds(base + off, sz)], osem)

    gather = [None] * NBUF
    outcp = [None] * NBUF
    out_waited = [True] * NBUF
    for j in range(min(NBUF - 1, NCH)):
        gather[j % NBUF] = start_gather(j)
    for j in range(NCH):
        b = j % NBUF
        gather[b].wait()
        outcp[b] = start_out(j)
        out_waited[b] = False
        nj = j + NBUF - 1
        if nj < NCH:
            nb = nj % NBUF
            if not out_waited[nb]:
                outcp[nb].wait()
                out_waited[nb] = True
            gather[nb] = start_gather(nj)
    for b in range(NBUF):
        if not out_waited[b]:
            outcp[b].wait()


def kernel(x, table):
    idx = x.reshape(NW, BPW).astype(jnp.int32)
    mesh = plsc.VectorSubcoreMesh(core_axis_name="c", subcore_axis_name="s")
    out = pl.kernel(
        _pe_body,
        mesh=mesh,
        out_type=jax.ShapeDtypeStruct((B, D), jnp.float32),
        scratch_types=[
            pltpu.VMEM((BPW,), jnp.int32),
            pltpu.VMEM((NBUF, C, D), jnp.float32),
            pltpu.SemaphoreType.DMA,
            pltpu.SemaphoreType.DMA,
        ],
    )(idx, table)
    return out.reshape(x.shape[0], x.shape[1], D)


# speedup vs baseline: 1.0377x; 1.0377x over previous
import jax
import jax.numpy as jnp
from jax import lax
from jax.experimental import pallas as pl
from jax.experimental.pallas import tpu as pltpu
from jax.experimental.pallas import tpu_sc as plsc

D = 1024
NC = 2
NS = 16
NW = NC * NS
B = 4 * 8192
BPW = B // NW
C = 40
CHUNKS = [(i * C, C) for i in range(BPW // C)] + [(BPW - BPW % C, BPW % C)]
NCH = len(CHUNKS)
NBUF = 3


def _pe_body(idx_hbm, table_hbm, out_hbm, idx_v, rows_v, gsem, osem):
    wid = lax.axis_index("s") * NC + lax.axis_index("c")
    base = wid * BPW
    pltpu.sync_copy(idx_hbm.at[wid], idx_v)

    def start_gather(j):
        off, sz = CHUNKS[j]
        slot = j % NBUF
        dst = rows_v.at[slot] if sz == C else rows_v.at[slot].at[pl.ds(0, sz)]
        return pltpu.async_copy(
            table_hbm.at[idx_v.at[pl.ds(off, sz)]], dst, gsem)

    def start_out(j):
        off, sz = CHUNKS[j]
        slot = j % NBUF
        src = rows_v.at[slot] if sz == C else rows_v.at[slot].at[pl.ds(0, sz)]
        return pltpu.async_copy(
            src, out_hbm.at[pl.ds(base + off, sz)], osem)

    gather = [None] * NBUF
    outcp = [None] * NBUF
    out_waited = [True] * NBUF
    for j in range(min(NBUF - 1, NCH)):
        gather[j % NBUF] = start_gather(j)
    for j in range(NCH):
        b = j % NBUF
        gather[b].wait()
        outcp[b] = start_out(j)
        out_waited[b] = False
        nj = j + NBUF - 1
        if nj < NCH:
            nb = nj % NBUF
            if not out_waited[nb]:
                outcp[nb].wait()
                out_waited[nb] = True
            gather[nb] = start_gather(nj)
    for b in range(NBUF):
        if not out_waited[b]:
            outcp[b].wait()


def kernel(x, table):
    idx = x.reshape(NW, BPW).astype(jnp.int32)
    mesh = plsc.VectorSubcoreMesh(core_axis_name="c", subcore_axis_name="s")
    out = pl.kernel(
        _pe_body,
        mesh=mesh,
        out_type=jax.ShapeDtypeStruct((B, D), jnp.float32),
        scratch_types=[
            pltpu.VMEM((BPW,), jnp.int32),
            pltpu.VMEM((NBUF, C, D), jnp.float32),
            pltpu.SemaphoreType.DMA,
            pltpu.SemaphoreType.DMA,
        ],
    )(idx, table)
    return out.reshape(x.shape[0], x.shape[1], D)
